# Initial kernel scaffold; baseline (speedup 1.0000x reference)
import jax
import jax.numpy as jnp
from jax.experimental import pallas as pl


def kernel(input):
    raise NotImplementedError("write your pallas kernel here")


# speedup vs baseline: 7.6561x; 7.6561x over previous
import functools

import jax
import jax.numpy as jnp
from jax import lax
from jax.experimental import pallas as pl
from jax.experimental.pallas import tpu as pltpu
from jax.experimental.pallas import tpu_sc as plsc

B = 16
C = 3
N = 4096
L = 16
CHUNKS = N // L


def _fps_body(in_hbm, out_hbm, x_ref, y_ref, z_ref, d_ref, ox_ref, oy_ref, oz_ref):
    cid = lax.axis_index("c")
    sid = lax.axis_index("s")

    @pl.when(sid < B // 2)
    def _run():
        b = cid * (B // 2) + sid
        base = b * (C * N)

        pltpu.sync_copy(in_hbm.at[pl.ds(base + 0 * N, N)], x_ref)
        pltpu.sync_copy(in_hbm.at[pl.ds(base + 1 * N, N)], y_ref)
        pltpu.sync_copy(in_hbm.at[pl.ds(base + 2 * N, N)], z_ref)

        lanes = lax.iota(jnp.int32, L)
        lane0 = lanes == 0
        big = jnp.full((L,), 1e10, jnp.float32)

        def init_body(c, carry):
            d_ref[pl.ds(c * L, L)] = big
            return carry

        lax.fori_loop(0, CHUNKS, init_body, 0, unroll=8)

        def step(s, bestv):
            cxv = plsc.load_gather(x_ref, [bestv])
            cyv = plsc.load_gather(y_ref, [bestv])
            czv = plsc.load_gather(z_ref, [bestv])
            sv = jnp.full((L,), s, jnp.int32)
            plsc.store_scatter(ox_ref, [sv], cxv, mask=lane0)
            plsc.store_scatter(oy_ref, [sv], cyv, mask=lane0)
            plsc.store_scatter(oz_ref, [sv], czv, mask=lane0)

            def chunk(c, carry):
                maxv, maxi = carry
                off = c * L
                xv = x_ref[pl.ds(off, L)]
                yv = y_ref[pl.ds(off, L)]
                zv = z_ref[pl.ds(off, L)]
                dx = xv - cxv
                dy = yv - cyv
                dz = zv - czv
                dd = (dx * dx + dz * dz) + dy * dy
                nd = jnp.minimum(d_ref[pl.ds(off, L)], dd)
                d_ref[pl.ds(off, L)] = nd
                idxv = off + lanes
                better = nd > maxv
                maxv = jnp.where(better, nd, maxv)
                maxi = jnp.where(better, idxv, maxi)
                return maxv, maxi

            neg1 = jnp.full((L,), -1.0, jnp.float32)
            zeroi = jnp.zeros((L,), jnp.int32)
            maxv, maxi = lax.fori_loop(0, CHUNKS, chunk, (neg1, zeroi), unroll=8)

            m = jnp.max(maxv)
            cand = jnp.where(maxv == m, maxi, jnp.int32(2**31 - 1))
            best = jnp.min(cand)
            return jnp.full((L,), best, jnp.int32)

        lax.fori_loop(0, N, step, jnp.zeros((L,), jnp.int32))

        pltpu.sync_copy(ox_ref, out_hbm.at[pl.ds(base + 0 * N, N)])
        pltpu.sync_copy(oy_ref, out_hbm.at[pl.ds(base + 1 * N, N)])
        pltpu.sync_copy(oz_ref, out_hbm.at[pl.ds(base + 2 * N, N)])


@jax.jit
def _fps(x):
    mesh = plsc.VectorSubcoreMesh(core_axis_name="c", subcore_axis_name="s", num_cores=2, num_subcores=16)
    f = functools.partial(
        pl.kernel,
        out_type=jax.ShapeDtypeStruct((B * C * N,), jnp.float32),
        mesh=mesh,
        scratch_types=[
            pltpu.VMEM((N,), jnp.float32),
            pltpu.VMEM((N,), jnp.float32),
            pltpu.VMEM((N,), jnp.float32),
            pltpu.VMEM((N,), jnp.float32),
            pltpu.VMEM((N,), jnp.float32),
            pltpu.VMEM((N,), jnp.float32),
            pltpu.VMEM((N,), jnp.float32),
        ],
        name="fps_sc",
        compiler_params=pltpu.CompilerParams(needs_layout_passes=False),
    )(_fps_body)
    return f(x.reshape(-1)).reshape(B, C, N)


def kernel(input):
    return _fps(input)


# speedup vs baseline: 20.2346x; 2.6429x over previous
import functools

import jax
import jax.numpy as jnp
from jax import lax
from jax.experimental import pallas as pl
from jax.experimental.pallas import tpu as pltpu
from jax.experimental.pallas import tpu_sc as plsc

B = 16
C = 3
N = 4096
L = 16
CHUNKS = N // L


def _fps_body(in_hbm, out_hbm, x_ref, y_ref, z_ref, d_ref, ox_ref, oy_ref, oz_ref):
    cid = lax.axis_index("c")
    sid = lax.axis_index("s")

    @pl.when(sid < B // 2)
    def _run():
        b = cid * (B // 2) + sid
        base = b * (C * N)

        pltpu.sync_copy(in_hbm.at[pl.ds(base + 0 * N, N)], x_ref)
        pltpu.sync_copy(in_hbm.at[pl.ds(base + 1 * N, N)], y_ref)
        pltpu.sync_copy(in_hbm.at[pl.ds(base + 2 * N, N)], z_ref)

        lanes = lax.iota(jnp.int32, L)
        lane0 = lanes == 0
        big = jnp.full((L,), 1e10, jnp.float32)

        def init_body(c, carry):
            d_ref[pl.ds(c * L, L)] = big
            return carry

        lax.fori_loop(0, CHUNKS, init_body, 0, unroll=8)

        def step(s, bestv):
            cxv = plsc.load_gather(x_ref, [bestv])
            cyv = plsc.load_gather(y_ref, [bestv])
            czv = plsc.load_gather(z_ref, [bestv])
            sv = jnp.full((L,), s, jnp.int32)
            plsc.store_scatter(ox_ref, [sv], cxv, mask=lane0)
            plsc.store_scatter(oy_ref, [sv], cyv, mask=lane0)
            plsc.store_scatter(oz_ref, [sv], czv, mask=lane0)

            neg1 = jnp.full((L,), -1.0, jnp.float32)
            zeroi = jnp.zeros((L,), jnp.int32)

            @plsc.parallel_loop(0, N, step=L, unroll=8, carry=(neg1, zeroi))
            def chunk(off, carry):
                maxv, maxi = carry
                xv = x_ref[pl.ds(off, L)]
                yv = y_ref[pl.ds(off, L)]
                zv = z_ref[pl.ds(off, L)]
                dx = xv - cxv
                dy = yv - cyv
                dz = zv - czv
                dd = (dx * dx + dz * dz) + dy * dy
                nd = jnp.minimum(d_ref[pl.ds(off, L)], dd)
                d_ref[pl.ds(off, L)] = nd
                idxv = off + lanes
                better = (nd > maxv) | ((nd == maxv) & (idxv < maxi))
                maxv = jnp.where(better, nd, maxv)
                maxi = jnp.where(better, idxv, maxi)
                return maxv, maxi

            maxv, maxi = chunk

            m = jnp.max(maxv)
            cand = jnp.where(maxv == m, maxi, jnp.int32(2**31 - 1))
            best = jnp.min(cand)
            return jnp.full((L,), best, jnp.int32)

        lax.fori_loop(0, N, step, jnp.zeros((L,), jnp.int32))

        pltpu.sync_copy(ox_ref, out_hbm.at[pl.ds(base + 0 * N, N)])
        pltpu.sync_copy(oy_ref, out_hbm.at[pl.ds(base + 1 * N, N)])
        pltpu.sync_copy(oz_ref, out_hbm.at[pl.ds(base + 2 * N, N)])


@jax.jit
def _fps(x):
    mesh = plsc.VectorSubcoreMesh(core_axis_name="c", subcore_axis_name="s", num_cores=2, num_subcores=16)
    f = functools.partial(
        pl.kernel,
        out_type=jax.ShapeDtypeStruct((B * C * N,), jnp.float32),
        mesh=mesh,
        scratch_types=[
            pltpu.VMEM((N,), jnp.float32),
            pltpu.VMEM((N,), jnp.float32),
            pltpu.VMEM((N,), jnp.float32),
            pltpu.VMEM((N,), jnp.float32),
            pltpu.VMEM((N,), jnp.float32),
            pltpu.VMEM((N,), jnp.float32),
            pltpu.VMEM((N,), jnp.float32),
        ],
        name="fps_sc",
        compiler_params=pltpu.CompilerParams(needs_layout_passes=False),
    )(_fps_body)
    return f(x.reshape(-1)).reshape(B, C, N)


def kernel(input):
    return _fps(input)
